# baseline (speedup 1.0000x reference)
import functools

import jax
import jax.numpy as jnp
from jax import lax
from jax.experimental import pallas as pl
from jax.experimental.pallas import tpu as pltpu
from jax.experimental.pallas import tpu_sc as plsc

_N_TOKENS = 8192
_DIM = 2048
_N_EXPERTS = 64
_GROUP_SIZE = 32
_TC_BLOCK = 2048

_NC = 2
_NS = 16
_L = 16
_NW = _NC * _NS
_TPW = _N_TOKENS // _NW
_CHUNKS = _TPW // _L

_NEG = -3.0e38


def _matmul_block(w_ref, x_ref, s_ref):
    s_ref[...] = lax.dot_general(
        w_ref[...],
        x_ref[...],
        dimension_numbers=(((1,), (1,)), ((), ())),
        preferred_element_type=jnp.float32,
    )


def _scores_t_tc(x, router_w):
    n = x.shape[0]
    return pl.pallas_call(
        _matmul_block,
        grid=(n // _TC_BLOCK,),
        in_specs=[
            pl.BlockSpec((_N_EXPERTS, _DIM), lambda i: (0, 0)),
            pl.BlockSpec((_TC_BLOCK, _DIM), lambda i: (i, 0)),
        ],
        out_specs=pl.BlockSpec((_N_EXPERTS, _TC_BLOCK), lambda i: (0, i)),
        out_shape=jax.ShapeDtypeStruct((_N_EXPERTS, n), jnp.float32),
        compiler_params=pltpu.CompilerParams(
            dimension_semantics=("arbitrary",),
        ),
    )(router_w, x)


def _route_body(scores_hbm, w_hbm, i_hbm, buf, wbuf, ibuf):
    wid = lax.axis_index("s") * _NC + lax.axis_index("c")
    base = wid * _TPW
    pltpu.sync_copy(scores_hbm.at[:, pl.ds(base, _TPW)], buf)

    def chunk(c, carry):
        off = c * _L
        z = jnp.zeros((_L,), jnp.float32)
        neg = jnp.full((_L,), _NEG, jnp.float32)
        zero_i = jnp.zeros((_L,), jnp.int32)
        m1 = [neg, neg]
        m2 = [neg, neg]
        i1 = [zero_i, zero_i]
        i2 = [zero_i, zero_i]
        for e in range(_N_EXPERTS):
            g = e // _GROUP_SIZE
            v = buf[e, pl.ds(off, _L)]
            z = z + jnp.exp(v)
            e_splat = jnp.full((_L,), e, jnp.int32)
            gt1 = v > m1[g]
            gt2 = v > m2[g]
            new_m2 = jnp.where(gt1, m1[g], jnp.where(gt2, v, m2[g]))
            new_i2 = jnp.where(gt1, i1[g], jnp.where(gt2, e_splat, i2[g]))
            m1[g] = jnp.where(gt1, v, m1[g])
            i1[g] = jnp.where(gt1, e_splat, i1[g])
            m2[g] = new_m2
            i2[g] = new_i2
        sel0 = m1[0] >= m1[1]
        b1 = jnp.where(sel0, m1[0], m1[1])
        b2 = jnp.where(sel0, m2[0], m2[1])
        j1 = jnp.where(sel0, i1[0], i1[1])
        j2 = jnp.where(sel0, i2[0], i2[1])
        wbuf[0, pl.ds(off, _L)] = jnp.exp(b1) / z
        wbuf[1, pl.ds(off, _L)] = jnp.exp(b2) / z
        ibuf[0, pl.ds(off, _L)] = j1
        ibuf[1, pl.ds(off, _L)] = j2
        return carry

    lax.fori_loop(0, _CHUNKS, chunk, 0)

    pltpu.sync_copy(wbuf, w_hbm.at[:, pl.ds(base, _TPW)])
    pltpu.sync_copy(ibuf, i_hbm.at[:, pl.ds(base, _TPW)])


_route_sc = functools.partial(
    pl.kernel,
    out_type=[
        jax.ShapeDtypeStruct((2, _N_TOKENS), jnp.float32),
        jax.ShapeDtypeStruct((2, _N_TOKENS), jnp.int32),
    ],
    mesh=plsc.VectorSubcoreMesh(core_axis_name="c", subcore_axis_name="s"),
    scratch_types=[
        pltpu.VMEM((_N_EXPERTS, _TPW), jnp.float32),
        pltpu.VMEM((2, _TPW), jnp.float32),
        pltpu.VMEM((2, _TPW), jnp.int32),
    ],
)(_route_body)


@jax.jit
def kernel(x, router_w):
    scores_t = _scores_t_tc(x, router_w)
    w_t, i_t = _route_sc(scores_t)
    return w_t.T, i_t.T


# speedup vs baseline: 1.1540x; 1.1540x over previous
import jax
import jax.numpy as jnp
from jax.experimental import pallas as pl
from jax.experimental.pallas import tpu as pltpu

_DIM = 2048
_N_EXPERTS = 64
_N_GROUPS = 2
_GROUP_SIZE = _N_EXPERTS // _N_GROUPS
_BLOCK = 2048

_KEY_MIN = -2147483647 - 1


def _to_key(s):
    u = jax.lax.bitcast_convert_type(s, jnp.int32)
    return jnp.where(u < 0, u ^ jnp.int32(0x7FFFFFFF), u)


def _from_key(k):
    u = jnp.where(k < 0, k ^ jnp.int32(0x7FFFFFFF), k)
    return jax.lax.bitcast_convert_type(u, jnp.float32)


def _gate_block(x_ref, wt_ref, w_out_ref, i_out_ref):
    s = jnp.dot(x_ref[...], wt_ref[...], preferred_element_type=jnp.float32)

    lane = jax.lax.broadcasted_iota(jnp.int32, s.shape, 1)
    key = _to_key(s)

    km0 = jnp.where(lane < _GROUP_SIZE, key, _KEY_MIN)
    km1 = jnp.where(lane >= _GROUP_SIZE, key, _KEY_MIN)
    kg0 = jnp.max(km0, axis=-1, keepdims=True)
    kg1 = jnp.max(km1, axis=-1, keepdims=True)
    k1 = jnp.maximum(kg0, kg1)
    i1 = jnp.min(jnp.where(key == k1, lane, _N_EXPERTS), axis=-1, keepdims=True)

    drop = lane == i1
    kd0 = jnp.max(jnp.where(drop, _KEY_MIN, km0), axis=-1, keepdims=True)
    kd1 = jnp.max(jnp.where(drop, _KEY_MIN, km1), axis=-1, keepdims=True)
    k2 = jnp.where(kg1 > kg0, kd1, kd0)
    i2 = jnp.min(
        jnp.where(jnp.logical_or(drop, key != k2), _N_EXPERTS, lane),
        axis=-1,
        keepdims=True,
    )

    m = _from_key(k1)
    z = jnp.sum(jnp.exp(s - m), axis=-1, keepdims=True)
    w1 = jnp.exp(_from_key(k1) - m) / z
    w2 = jnp.exp(_from_key(k2) - m) / z

    w_out_ref[...] = jnp.concatenate([w1, w2], axis=-1)
    i_out_ref[...] = jnp.concatenate([i1, i2], axis=-1)


@jax.jit
def kernel(x, router_w):
    n = x.shape[0]
    grid = (n // _BLOCK,)
    wt = router_w.T
    weights, indices = pl.pallas_call(
        _gate_block,
        grid=grid,
        in_specs=[
            pl.BlockSpec((_BLOCK, _DIM), lambda i: (i, 0)),
            pl.BlockSpec((_DIM, _N_EXPERTS), lambda i: (0, 0)),
        ],
        out_specs=[
            pl.BlockSpec((_BLOCK, 2), lambda i: (i, 0)),
            pl.BlockSpec((_BLOCK, 2), lambda i: (i, 0)),
        ],
        out_shape=[
            jax.ShapeDtypeStruct((n, 2), jnp.float32),
            jax.ShapeDtypeStruct((n, 2), jnp.int32),
        ],
        compiler_params=pltpu.CompilerParams(
            dimension_semantics=("parallel",),
        ),
    )(x, wt)
    return weights, indices
